# baseline (speedup 1.0000x reference)
import jax
import jax.numpy as jnp
from jax.experimental import pallas as pl
from jax.experimental.pallas import tpu as pltpu

_E = 5
_K = 3


def _gating_kernel(x_ref, wg_ref, bg_ref, p_ref):
    logits = jnp.dot(x_ref[...], wg_ref[...],
                     preferred_element_type=jnp.float32) + bg_ref[...]
    g = jax.nn.softmax(logits, axis=-1)
    cols = [g[:, e:e + 1] for e in range(_E)]
    ps = []
    for e in range(_E):
        cnt = jnp.zeros_like(cols[e])
        for e2 in range(_E):
            if e2 == e:
                continue
            if e2 < e:
                beats = cols[e2] >= cols[e]
            else:
                beats = cols[e2] > cols[e]
            cnt = cnt + beats.astype(jnp.float32)
        ps.append(jnp.where(cnt < float(_K), cols[e], 0.0))
    p_ref[...] = jnp.concatenate(ps, axis=1)


def _combine_kernel(p_ref, x_ref, we_ref, be_ref, o_ref):
    x = x_ref[...]
    p = p_ref[...]
    acc = jnp.zeros((x.shape[0], o_ref.shape[1]), jnp.float32)
    for e in range(_E):
        y_e = jnp.dot(x, we_ref[e], preferred_element_type=jnp.float32)
        acc = acc + p[:, e:e + 1] * (y_e + be_ref[e][None, :])
    o_ref[...] = acc


def kernel(x, Wg, bg, We, be):
    B, D = x.shape
    E, _, H = We.shape
    bg2 = bg.reshape(1, E)

    Tg = 1024
    p = pl.pallas_call(
        _gating_kernel,
        grid=(B // Tg,),
        in_specs=[
            pl.BlockSpec((Tg, D), lambda t: (t, 0)),
            pl.BlockSpec((D, E), lambda t: (0, 0)),
            pl.BlockSpec((1, E), lambda t: (0, 0)),
        ],
        out_specs=pl.BlockSpec((Tg, E), lambda t: (t, 0)),
        out_shape=jax.ShapeDtypeStruct((B, E), jnp.float32),
    )(x, Wg, bg2)

    Tb = 1024
    Hb = 512
    xb = x.astype(jnp.bfloat16)
    web = We.astype(jnp.bfloat16)
    return pl.pallas_call(
        _combine_kernel,
        grid=(H // Hb, B // Tb),
        in_specs=[
            pl.BlockSpec((Tb, E), lambda h, t: (t, 0)),
            pl.BlockSpec((Tb, D), lambda h, t: (t, 0)),
            pl.BlockSpec((E, D, Hb), lambda h, t: (0, 0, h)),
            pl.BlockSpec((E, Hb), lambda h, t: (0, h)),
        ],
        out_specs=pl.BlockSpec((Tb, Hb), lambda h, t: (t, h)),
        out_shape=jax.ShapeDtypeStruct((B, H), jnp.float32),
        compiler_params=pltpu.CompilerParams(vmem_limit_bytes=67108864),
    )(p, xb, web, be)


# speedup vs baseline: 1.1563x; 1.1563x over previous
import jax
import jax.numpy as jnp
from jax.experimental import pallas as pl
from jax.experimental.pallas import tpu as pltpu

_E = 5
_K = 3


def _gating_kernel(x_ref, wg_ref, bg_ref, p_ref):
    logits = jnp.dot(x_ref[...], wg_ref[...],
                     preferred_element_type=jnp.float32) + bg_ref[...]
    g = jax.nn.softmax(logits, axis=-1)
    cols = [g[:, e:e + 1] for e in range(_E)]
    ps = []
    for e in range(_E):
        cnt = jnp.zeros_like(cols[e])
        for e2 in range(_E):
            if e2 == e:
                continue
            if e2 < e:
                beats = cols[e2] >= cols[e]
            else:
                beats = cols[e2] > cols[e]
            cnt = cnt + beats.astype(jnp.float32)
        ps.append(jnp.where(cnt < float(_K), cols[e], 0.0))
    p_ref[...] = jnp.concatenate(ps, axis=1)


def _combine_kernel(p_ref, x_ref, we_ref, be_ref, o_ref):
    x = x_ref[...]
    p = p_ref[...]
    acc = jnp.dot(p, be_ref[...], preferred_element_type=jnp.float32)
    for e in range(_E):
        y_e = jnp.dot(x, we_ref[e], preferred_element_type=jnp.float32)
        acc = acc + p[:, e:e + 1] * y_e
    o_ref[...] = acc


def kernel(x, Wg, bg, We, be):
    B, D = x.shape
    E, _, H = We.shape
    bg2 = bg.reshape(1, E)

    Tg = 1024
    p = pl.pallas_call(
        _gating_kernel,
        grid=(B // Tg,),
        in_specs=[
            pl.BlockSpec((Tg, D), lambda t: (t, 0)),
            pl.BlockSpec((D, E), lambda t: (0, 0)),
            pl.BlockSpec((1, E), lambda t: (0, 0)),
        ],
        out_specs=pl.BlockSpec((Tg, E), lambda t: (t, 0)),
        out_shape=jax.ShapeDtypeStruct((B, E), jnp.float32),
    )(x, Wg, bg2)

    Tb = 512
    Hb = 512
    xb = x
    web = We
    return pl.pallas_call(
        _combine_kernel,
        grid=(H // Hb, B // Tb),
        in_specs=[
            pl.BlockSpec((Tb, E), lambda h, t: (t, 0)),
            pl.BlockSpec((Tb, D), lambda h, t: (t, 0)),
            pl.BlockSpec((E, D, Hb), lambda h, t: (0, 0, h)),
            pl.BlockSpec((E, Hb), lambda h, t: (0, h)),
        ],
        out_specs=pl.BlockSpec((Tb, Hb), lambda h, t: (t, h)),
        out_shape=jax.ShapeDtypeStruct((B, H), jnp.float32),
        compiler_params=pltpu.CompilerParams(vmem_limit_bytes=67108864),
    )(p, xb, web, be)
